# Initial kernel scaffold; baseline (speedup 1.0000x reference)
import jax
import jax.numpy as jnp
from jax.experimental import pallas as pl


def kernel(x, weight, n):
    raise NotImplementedError("write your pallas kernel here")


# speedup vs baseline: 19.3100x; 19.3100x over previous
import functools

import jax
import jax.numpy as jnp
import numpy as np
from jax.experimental import pallas as pl
from jax.experimental.pallas import tpu as pltpu

_NP = 20
_S = 2048
_CB = 128


def _fspool_kernel(jk_ref, n_ref, x_ref, w_ref, pooled_ref, perm_ref):
    b = pl.program_id(0)
    n = n_ref[b]
    denom = jnp.maximum(n.astype(jnp.float32) - 1.0, 1.0)

    xt = x_ref[0].T

    sio_i = jax.lax.broadcasted_iota(jnp.int32, (_S, 1), 0)
    sio_f = sio_i.astype(jnp.float32)
    maskc = sio_f <= denom

    key = jnp.where(maskc, -xt, 99999.0 - xt)
    idx = jax.lax.broadcasted_iota(jnp.int32, (_S, _CB), 0).astype(jnp.float32)

    def stage(t, carry):
        key, idx = carry
        j = jk_ref[t, 0]
        k = jk_ref[t, 1]
        bitj0 = (sio_i & j) == 0
        bitk0 = (sio_i & k) == 0
        tmn = bitj0 != bitk0
        kp = jnp.where(bitj0, pltpu.roll(key, _S - j, 0), pltpu.roll(key, j, 0))
        ip = jnp.where(bitj0, pltpu.roll(idx, _S - j, 0), pltpu.roll(idx, j, 0))
        p_lt = (kp < key) | ((kp == key) & (ip < idx))
        take_p = p_lt != tmn
        return (jnp.where(take_p, kp, key), jnp.where(take_p, ip, idx))

    nstages = jk_ref.shape[0]
    key, idx = jax.lax.fori_loop(0, nstages, stage, (key, idx))

    perm_ref[0] = idx.astype(jnp.int32).T

    xs = -key
    sizes = jnp.minimum(sio_f / denom, 1.0)
    findex = float(_NP) * sizes
    fidx = jnp.floor(findex)
    frac = findex - fidx
    lane = jax.lax.broadcasted_iota(jnp.int32, (_S, _CB), 1).astype(jnp.float32)
    coef = jnp.where(lane == fidx, 1.0 - frac, 0.0)
    coef = coef + jnp.where(lane == jnp.minimum(fidx + 1.0, float(_NP)), frac, 0.0)
    coef = jnp.where(maskc, coef, 0.0)
    t = jax.lax.dot_general(xs, coef, (((0,), (0,)), ((), ())),
                            preferred_element_type=jnp.float32)
    pooled_ref[0, 0, 0] = jnp.sum(t * w_ref[...], axis=1)


def _stage_table():
    rows = []
    k = 2
    while k <= _S:
        j = k // 2
        while j >= 1:
            rows.append((j, k))
            j //= 2
        k *= 2
    return np.asarray(rows, dtype=np.int32)


@jax.jit
def kernel(x, weight, n):
    B, C, S = x.shape
    jk = jnp.asarray(_stage_table())
    wpad = jnp.zeros((C, _CB), jnp.float32).at[:, : _NP + 1].set(weight)
    grid = (B, C // _CB)
    pooled, perm = pl.pallas_call(
        _fspool_kernel,
        grid=grid,
        in_specs=[
            pl.BlockSpec(memory_space=pltpu.SMEM),
            pl.BlockSpec(memory_space=pltpu.SMEM),
            pl.BlockSpec((1, _CB, S), lambda b, c: (b, c, 0)),
            pl.BlockSpec((_CB, _CB), lambda b, c: (c, 0)),
        ],
        out_specs=[
            pl.BlockSpec((1, 1, 1, _CB), lambda b, c: (b, c, 0, 0)),
            pl.BlockSpec((1, _CB, S), lambda b, c: (b, c, 0)),
        ],
        out_shape=[
            jax.ShapeDtypeStruct((B, C // _CB, 1, _CB), jnp.float32),
            jax.ShapeDtypeStruct((B, C, S), jnp.int32),
        ],
    )(jk, n.astype(jnp.int32), x, wpad)
    return pooled.reshape(B, C), perm


# speedup vs baseline: 56.2247x; 2.9117x over previous
import functools

import jax
import jax.numpy as jnp
import numpy as np
from jax.experimental import pallas as pl
from jax.experimental.pallas import tpu as pltpu

_NP = 20
_S = 2048
_CB = 128


def _fspool_kernel(jk_ref, n_ref, x_ref, w_ref, pooled_ref, perm_ref,
                   pk_ref, pi_ref):
    b = pl.program_id(0)
    n = n_ref[b]
    denom = jnp.maximum(n.astype(jnp.float32) - 1.0, 1.0)

    xt = x_ref[0].T

    sio_i = jax.lax.broadcasted_iota(jnp.int32, (_S, 1), 0)
    sio_f = sio_i.astype(jnp.float32)
    maskc = sio_f <= denom

    key = jnp.where(maskc, -xt, 99999.0 - xt)
    idx = jax.lax.broadcasted_iota(jnp.int32, (_S, _CB), 0).astype(jnp.float32)

    def _partner(x, j):
        if j >= 8:
            x4 = x.reshape(_S // (2 * j), 2, j, _CB)
            x4 = jnp.concatenate([x4[:, 1:], x4[:, :1]], axis=1)
            return x4.reshape(_S, _CB)
        bitj0 = (sio_i & j) == 0
        return jnp.where(bitj0, pltpu.roll(x, _S - j, 0), pltpu.roll(x, j, 0))

    def stage(t, carry):
        key, idx = carry
        lj = jk_ref[t, 0]
        k = jk_ref[t, 1]
        j = jnp.left_shift(1, lj)
        for a in range(11):

            @pl.when(lj == a)
            def _(key=key, idx=idx, j=1 << a):
                pk_ref[...] = _partner(key, j)
                pi_ref[...] = _partner(idx, j)

        kp = pk_ref[...]
        ip = pi_ref[...]
        bitj0 = (sio_i & j) == 0
        bitk0 = (sio_i & k) == 0
        tmn = bitj0 != bitk0
        p_lt = (kp < key) | ((kp == key) & (ip < idx))
        take_p = p_lt != tmn
        return (jnp.where(take_p, kp, key), jnp.where(take_p, ip, idx))

    nstages = jk_ref.shape[0]
    key, idx = jax.lax.fori_loop(0, nstages, stage, (key, idx))

    perm_ref[0] = idx.astype(jnp.int32).T

    xs = -key
    sizes = jnp.minimum(sio_f / denom, 1.0)
    findex = float(_NP) * sizes
    fidx = jnp.floor(findex)
    frac = findex - fidx
    lane = jax.lax.broadcasted_iota(jnp.int32, (_S, _CB), 1).astype(jnp.float32)
    coef = jnp.where(lane == fidx, 1.0 - frac, 0.0)
    coef = coef + jnp.where(lane == jnp.minimum(fidx + 1.0, float(_NP)), frac, 0.0)
    coef = jnp.where(maskc, coef, 0.0)
    t = jax.lax.dot_general(xs, coef, (((0,), (0,)), ((), ())),
                            preferred_element_type=jnp.float32)
    pooled_ref[0, 0, 0] = jnp.sum(t * w_ref[...], axis=1)


def _stage_table():
    rows = []
    k = 2
    while k <= _S:
        j = k // 2
        while j >= 1:
            rows.append((int(np.log2(j)), k))
            j //= 2
        k *= 2
    return np.asarray(rows, dtype=np.int32)


@jax.jit
def kernel(x, weight, n):
    B, C, S = x.shape
    jk = jnp.asarray(_stage_table())
    wpad = jnp.zeros((C, _CB), jnp.float32).at[:, : _NP + 1].set(weight)
    grid = (B, C // _CB)
    pooled, perm = pl.pallas_call(
        _fspool_kernel,
        grid=grid,
        in_specs=[
            pl.BlockSpec(memory_space=pltpu.SMEM),
            pl.BlockSpec(memory_space=pltpu.SMEM),
            pl.BlockSpec((1, _CB, S), lambda b, c: (b, c, 0)),
            pl.BlockSpec((_CB, _CB), lambda b, c: (c, 0)),
        ],
        out_specs=[
            pl.BlockSpec((1, 1, 1, _CB), lambda b, c: (b, c, 0, 0)),
            pl.BlockSpec((1, _CB, S), lambda b, c: (b, c, 0)),
        ],
        out_shape=[
            jax.ShapeDtypeStruct((B, C // _CB, 1, _CB), jnp.float32),
            jax.ShapeDtypeStruct((B, C, S), jnp.int32),
        ],
        scratch_shapes=[
            pltpu.VMEM((_S, _CB), jnp.float32),
            pltpu.VMEM((_S, _CB), jnp.float32),
        ],
    )(jk, n.astype(jnp.int32), x, wpad)
    return pooled.reshape(B, C), perm


# speedup vs baseline: 140.3539x; 2.4963x over previous
import functools

import jax
import jax.numpy as jnp
import numpy as np
from jax.experimental import pallas as pl
from jax.experimental.pallas import tpu as pltpu

_NP = 20
_S = 2048
_CB = 128
_T = 64
_NT = _S // _T


def _fspool_kernel(n_ref, x_ref, w_ref, pooled_ref, perm_ref, kref, iref):
    b = pl.program_id(0)
    n = n_ref[b]
    denom = jnp.maximum(n.astype(jnp.float32) - 1.0, 1.0)

    xt = x_ref[0].T

    sio_f = jax.lax.broadcasted_iota(jnp.int32, (_S, 1), 0).astype(jnp.float32)
    maskc = sio_f <= denom

    kref[...] = jnp.where(maskc, -xt, 99999.0 - xt)
    iref[...] = jax.lax.broadcasted_iota(jnp.int32, (_S, _CB), 0).astype(
        jnp.float32)

    rio = jax.lax.broadcasted_iota(jnp.int32, (_T, 1), 0)

    def _partner(x, j, rows):
        if j >= 8:
            x4 = x.reshape(rows // (2 * j), 2, j, _CB)
            x4 = jnp.concatenate([x4[:, 1:], x4[:, :1]], axis=1)
            return x4.reshape(rows, _CB)
        bitj0 = (rio & j) == 0
        return jnp.where(bitj0, pltpu.roll(x, rows - j, 0),
                         pltpu.roll(x, j, 0))

    def _cswap(key, idx, j, tmn):
        kp = _partner(key, j, key.shape[0])
        ip = _partner(idx, j, idx.shape[0])
        p_lt = (kp < key) | ((kp == key) & (ip < idx))
        take_p = p_lt != tmn
        return jnp.where(take_p, kp, key), jnp.where(take_p, ip, idx)

    def _fused_tile_pass(stages):
        def body(t, _):
            base = t * _T
            key = kref[pl.ds(base, _T), :]
            idx = iref[pl.ds(base, _T), :]
            for j, k in stages:
                bitj0 = (rio & j) == 0
                if k <= _T // 2:
                    tmn = ((rio & k) == 0) != bitj0
                else:
                    dirbit = (base & k) != 0
                    tmn = bitj0 == dirbit
                key, idx = _cswap(key, idx, j, tmn)
            kref[pl.ds(base, _T), :] = key
            iref[pl.ds(base, _T), :] = idx
            return 0

        jax.lax.fori_loop(0, _NT, body, 0)

    def _far_stage(j, k):
        jt = j // _T

        def body(p, _):
            low = p & (jt - 1)
            t0 = (p - low) * 2 + low
            b0 = t0 * _T
            b1 = b0 + j
            ka = kref[pl.ds(b0, _T), :]
            ia = iref[pl.ds(b0, _T), :]
            kb = kref[pl.ds(b1, _T), :]
            ib = iref[pl.ds(b1, _T), :]
            tmn = (b0 & k) != 0
            b_first = (kb < ka) | ((kb == ka) & (ib < ia))
            swap = b_first != tmn
            kref[pl.ds(b0, _T), :] = jnp.where(swap, kb, ka)
            iref[pl.ds(b0, _T), :] = jnp.where(swap, ib, ia)
            kref[pl.ds(b1, _T), :] = jnp.where(swap, ka, kb)
            iref[pl.ds(b1, _T), :] = jnp.where(swap, ia, ib)
            return 0

        jax.lax.fori_loop(0, _NT // 2, body, 0)

    first = []
    k = 2
    while k <= _T:
        j = k // 2
        while j >= 1:
            first.append((j, k))
            j //= 2
        k *= 2
    _fused_tile_pass(first)

    k = 2 * _T
    while k <= _S:
        j = k // 2
        while j >= _T:
            _far_stage(j, k)
            j //= 2
        _fused_tile_pass([(j2, k) for j2 in (32, 16, 8, 4, 2, 1)])
        k *= 2

    idx = iref[...]
    perm_ref[0] = idx.astype(jnp.int32).T

    xs = -kref[...]
    sizes = jnp.minimum(sio_f / denom, 1.0)
    findex = float(_NP) * sizes
    fidx = jnp.floor(findex)
    frac = findex - fidx
    lane = jax.lax.broadcasted_iota(jnp.int32, (_S, _CB), 1).astype(jnp.float32)
    coef = jnp.where(lane == fidx, 1.0 - frac, 0.0)
    coef = coef + jnp.where(lane == jnp.minimum(fidx + 1.0, float(_NP)), frac, 0.0)
    coef = jnp.where(maskc, coef, 0.0)
    t = jax.lax.dot_general(xs, coef, (((0,), (0,)), ((), ())),
                            preferred_element_type=jnp.float32)
    pooled_ref[0, 0, 0] = jnp.sum(t * w_ref[...], axis=1)


@jax.jit
def kernel(x, weight, n):
    B, C, S = x.shape
    wpad = jnp.zeros((C, _CB), jnp.float32).at[:, : _NP + 1].set(weight)
    grid = (B, C // _CB)
    pooled, perm = pl.pallas_call(
        _fspool_kernel,
        grid=grid,
        in_specs=[
            pl.BlockSpec(memory_space=pltpu.SMEM),
            pl.BlockSpec((1, _CB, S), lambda b, c: (b, c, 0)),
            pl.BlockSpec((_CB, _CB), lambda b, c: (c, 0)),
        ],
        out_specs=[
            pl.BlockSpec((1, 1, 1, _CB), lambda b, c: (b, c, 0, 0)),
            pl.BlockSpec((1, _CB, S), lambda b, c: (b, c, 0)),
        ],
        out_shape=[
            jax.ShapeDtypeStruct((B, C // _CB, 1, _CB), jnp.float32),
            jax.ShapeDtypeStruct((B, C, S), jnp.int32),
        ],
        scratch_shapes=[
            pltpu.VMEM((_S, _CB), jnp.float32),
            pltpu.VMEM((_S, _CB), jnp.float32),
        ],
    )(n.astype(jnp.int32), x, wpad)
    return pooled.reshape(B, C), perm
